# baseline (speedup 1.0000x reference)
import dataclasses
import functools

import jax
import jax.numpy as jnp
from jax import lax
from jax.experimental import pallas as pl
from jax.experimental.pallas import tpu as pltpu
from jax.experimental.pallas import tpu_sc as plsc

N = 10000
E = 320000
NC = 2
NS = 16
CHUNK = 128
NCH = 157
EPT = NCH * CHUNK
EPAD = NS * EPT
NPAD = 10240
STRIP = NPAD // NS
BN = 1000


def _pair_swizzle(h):
    bn, dh = h.shape
    g = h.reshape(bn, dh // 32, 2, 16).transpose(0, 1, 3, 2)
    return g.reshape(bn, dh)


def _tc_in_body(dh, x_ref, w_ref, al_ref, ar_ref, f_ref, e_ref):
    f = jnp.dot(x_ref[...], w_ref[...], preferred_element_type=jnp.float32)
    fb = f.astype(jnp.bfloat16)
    f_ref[...] = jnp.stack([_pair_swizzle(fb[:, :dh]),
                            _pair_swizzle(fb[:, dh:])], axis=0)
    el = jnp.sum(f * al_ref[...], axis=1, keepdims=True)
    er = jnp.sum(f * ar_ref[...], axis=1, keepdims=True)
    e_ref[...] = jnp.concatenate([el, er], axis=1)


def _tc_feat_in(x, W, al, ar):
    n, din = x.shape
    dout = W.shape[1]
    dh = dout // 2
    return pl.pallas_call(
        functools.partial(_tc_in_body, dh),
        grid=(n // BN,),
        in_specs=[
            pl.BlockSpec((BN, din), lambda i: (i, 0)),
            pl.BlockSpec((din, dout), lambda i: (0, 0)),
            pl.BlockSpec((1, dout), lambda i: (0, 0)),
            pl.BlockSpec((1, dout), lambda i: (0, 0)),
        ],
        out_specs=[
            pl.BlockSpec((NC, BN, dh), lambda i: (0, i, 0)),
            pl.BlockSpec((BN, 2), lambda i: (i, 0)),
        ],
        out_shape=[
            jax.ShapeDtypeStruct((NC, n, dh), jnp.bfloat16),
            jax.ShapeDtypeStruct((n, 2), jnp.float32),
        ],
    )(x, W, al, ar)


def _tc_mid_body(dh, a_ref, dn_ref, b_ref, w_ref, al_ref, ar_ref,
                 f_ref, e_ref):
    s = jnp.concatenate([a_ref[0], a_ref[1]], axis=1)
    dn = dn_ref[0]
    h = s / (dn + 1e-9) + b_ref[...]
    f = jnp.dot(h, w_ref[...], preferred_element_type=jnp.float32)
    fb = f.astype(jnp.bfloat16)
    f_ref[...] = jnp.stack([_pair_swizzle(fb[:, :dh]),
                            _pair_swizzle(fb[:, dh:])], axis=0)
    el = jnp.sum(f * al_ref[...], axis=1, keepdims=True)
    er = jnp.sum(f * ar_ref[...], axis=1, keepdims=True)
    e_ref[...] = jnp.concatenate([el, er], axis=1)


def _tc_feat_mid(acc, den, b, W, al, ar):
    d1h = acc.shape[2]
    dout = W.shape[1]
    dh = dout // 2
    return pl.pallas_call(
        functools.partial(_tc_mid_body, dh),
        grid=(N // BN,),
        in_specs=[
            pl.BlockSpec((NC, BN, d1h), lambda i: (0, i, 0)),
            pl.BlockSpec((NC, BN, 1), lambda i: (0, i, 0)),
            pl.BlockSpec((1, 2 * d1h), lambda i: (0, 0)),
            pl.BlockSpec((2 * d1h, dout), lambda i: (0, 0)),
            pl.BlockSpec((1, dout), lambda i: (0, 0)),
            pl.BlockSpec((1, dout), lambda i: (0, 0)),
        ],
        out_specs=[
            pl.BlockSpec((NC, BN, dh), lambda i: (0, i, 0)),
            pl.BlockSpec((BN, 2), lambda i: (i, 0)),
        ],
        out_shape=[
            jax.ShapeDtypeStruct((NC, N, dh), jnp.bfloat16),
            jax.ShapeDtypeStruct((N, 2), jnp.float32),
        ],
    )(acc, den, b, W, al, ar)


def _tc_out_body(a_ref, dn_ref, b_ref, o_ref):
    s = jnp.concatenate([a_ref[0], a_ref[1]], axis=1)
    dn = dn_ref[0]
    h = s / (dn + 1e-9) + b_ref[...]
    m = jnp.max(h, axis=1, keepdims=True)
    ex = jnp.exp(h - m)
    o_ref[...] = ex / jnp.sum(ex, axis=1, keepdims=True)


def _tc_out(acc, den, b):
    dho = acc.shape[2]
    return pl.pallas_call(
        _tc_out_body,
        grid=(N // BN,),
        in_specs=[
            pl.BlockSpec((NC, BN, dho), lambda i: (0, i, 0)),
            pl.BlockSpec((NC, BN, 1), lambda i: (0, i, 0)),
            pl.BlockSpec((1, 2 * dho), lambda i: (0, 0)),
        ],
        out_specs=pl.BlockSpec((BN, 2 * dho), lambda i: (i, 0)),
        out_shape=jax.ShapeDtypeStruct((N, 2 * dho), jnp.float32),
    )(acc, den, b)


def _sc_body(dh, feata_hbm, featb_hbm, el_hbm, er_hbm, src_hbm, dst_hbm,
             acc_out, den_out, el_v, er_v, src_v, dst_v, ex_v, rows0_v,
             rows1_v, sbuf_v, acc_s, den_s, sem0, sem1):
    cid = lax.axis_index("c")
    sid = lax.axis_index("s")

    pltpu.sync_copy(el_hbm, el_v)
    pltpu.sync_copy(er_hbm, er_v)
    pltpu.sync_copy(src_hbm.at[sid], src_v)
    pltpu.sync_copy(dst_hbm.at[sid], dst_v)

    zero16 = jnp.zeros((16,), jnp.float32)

    @pl.loop(0, CHUNK)
    def _(e):
        for q in range(dh // 16):
            sbuf_v[e, pl.ds(q * 16, 16)] = zero16

    for q in range(CHUNK // 16):
        ex_v[pl.ds(q * 16, 16)] = zero16

    base = sid * STRIP
    for i in range(STRIP // CHUNK):
        pltpu.sync_copy(sbuf_v, acc_s.at[pl.ds(base + i * CHUNK, CHUNK)])
        pltpu.sync_copy(ex_v, den_s.at[pl.ds(base + i * CHUNK, CHUNK)])
    plsc.subcore_barrier()

    iota16 = lax.iota(jnp.int32, 16)
    gbase = sid * EPT

    def issue_gather(j, gbuf, sem):
        @pl.when(cid == 0)
        def _():
            pltpu.async_copy(feata_hbm.at[src_v.at[j]], gbuf, sem)

        @pl.when(cid != 0)
        def _():
            pltpu.async_copy(featb_hbm.at[src_v.at[j]], gbuf, sem)

    def compute_ex(j):
        @pl.loop(0, CHUNK, step=16)
        def _(k):
            s16 = src_v[j, pl.ds(k, 16)]
            d16 = dst_v[j, pl.ds(k, 16)]
            e16 = plsc.load_gather(el_v, [s16]) + plsc.load_gather(er_v, [d16])
            e16 = jnp.where(e16 >= 0.0, e16, e16 * 0.2)
            gid = gbase + j * CHUNK + k + iota16
            ex_v[pl.ds(k, 16)] = jnp.where(gid < E, jnp.exp(e16), 0.0)

    def wait_gather(gbuf, sem):
        pltpu.make_async_copy(feata_hbm.at[pl.ds(0, CHUNK)], gbuf, sem).wait()

    def scale(gbuf):
        @pl.loop(0, CHUNK, step=16)
        def _(k):
            w16 = ex_v[pl.ds(k, 16)]
            for i in range(16):
                w = w16[i]
                for q in range(dh // 32):
                    pr = gbuf[k + i, pl.ds(q * 32, 32)]
                    lo, hi = plsc.unpack(pr, format=plsc.PackFormat.INTERLEAVED)
                    sbuf_v[k + i, pl.ds(q * 32, 16)] = lo * w
                    sbuf_v[k + i, pl.ds(q * 32 + 16, 16)] = hi * w

    def scatter(j):
        pltpu.sync_copy(sbuf_v, acc_s.at[dst_v.at[j]], add=True)
        pltpu.sync_copy(ex_v, den_s.at[dst_v.at[j]], add=True)

    issue_gather(0, rows0_v, sem0)
    issue_gather(1, rows1_v, sem1)

    @pl.loop(0, NCH - 1, step=2)
    def _(j):
        compute_ex(j)
        wait_gather(rows0_v, sem0)
        scale(rows0_v)
        issue_gather(j + 2, rows0_v, sem0)
        scatter(j)

        compute_ex(j + 1)
        wait_gather(rows1_v, sem1)
        scale(rows1_v)
        issue_gather(j + 3, rows1_v, sem1)
        scatter(j + 1)

    compute_ex(NCH - 1)
    wait_gather(rows0_v, sem0)
    scale(rows0_v)
    scatter(NCH - 1)
    wait_gather(rows1_v, sem1)

    plsc.subcore_barrier()
    pltpu.sync_copy(acc_s.at[pl.ds(base, STRIP)],
                    acc_out.at[cid, pl.ds(base, STRIP)])
    pltpu.sync_copy(den_s.at[pl.ds(base, STRIP)],
                    den_out.at[cid, pl.ds(base, STRIP)])


def _sc_layer(feata, featb, el, er, src_t, dst_t):
    dh = feata.shape[1]
    mesh = plsc.VectorSubcoreMesh(core_axis_name="c", subcore_axis_name="s")
    cp = pltpu.CompilerParams(use_tc_tiling_on_sc=False)
    if "needs_layout_passes" in pltpu.CompilerParams.__dataclass_fields__:
        cp = dataclasses.replace(cp, needs_layout_passes=False)
    kern = pl.kernel(
        functools.partial(_sc_body, dh),
        compiler_params=cp,
        out_type=(jax.ShapeDtypeStruct((NC, NPAD, dh), jnp.float32),
                  jax.ShapeDtypeStruct((NC, NPAD), jnp.float32)),
        mesh=mesh,
        scratch_types=[
            pltpu.VMEM((N,), jnp.float32),
            pltpu.VMEM((N,), jnp.float32),
            pltpu.VMEM((NCH + 1, CHUNK), jnp.int32),
            pltpu.VMEM((NCH, CHUNK), jnp.int32),
            pltpu.VMEM((CHUNK,), jnp.float32),
            pltpu.VMEM((CHUNK, dh), jnp.bfloat16),
            pltpu.VMEM((CHUNK, dh), jnp.bfloat16),
            pltpu.VMEM((CHUNK, dh), jnp.float32),
            pltpu.VMEM_SHARED((NPAD, dh), jnp.float32),
            pltpu.VMEM_SHARED((NPAD,), jnp.float32),
            pltpu.SemaphoreType.DMA,
            pltpu.SemaphoreType.DMA,
        ],
    )
    return kern(feata, featb, el, er, src_t, dst_t)


def kernel(x, edge_index, W1, attn_l1, attn_r1, b1, W2, attn_l2, attn_r2, b2):
    src_t = jnp.pad(jnp.pad(edge_index[0], (0, EPAD - E)).reshape(NS, NCH, CHUNK),
                    ((0, 0), (0, 1), (0, 0)))
    dst_t = jnp.pad(edge_index[1], (0, EPAD - E)).reshape(NS, NCH, CHUNK)
    fs1, eler1 = _tc_feat_in(x, W1, attn_l1.reshape(1, -1),
                             attn_r1.reshape(1, -1))
    acc1, den1 = _sc_layer(fs1[0], fs1[1], eler1[:, 0], eler1[:, 1],
                           src_t, dst_t)
    fs2, eler2 = _tc_feat_mid(acc1, den1.reshape(NC, NPAD, 1), b1.reshape(1, -1), W2,
                              attn_l2.reshape(1, -1), attn_r2.reshape(1, -1))
    acc2, den2 = _sc_layer(fs2[0], fs2[1], eler2[:, 0], eler2[:, 1],
                           src_t, dst_t)
    return _tc_out(acc2, den2.reshape(NC, NPAD, 1), b2.reshape(1, -1))


# speedup vs baseline: 1.6138x; 1.6138x over previous
import dataclasses
import functools

import jax
import jax.numpy as jnp
from jax import lax
from jax.experimental import pallas as pl
from jax.experimental.pallas import tpu as pltpu
from jax.experimental.pallas import tpu_sc as plsc

N = 10000
E = 320000
NC = 2
NS = 16
CHUNK = 128
NCH = 157
EPT = NCH * CHUNK
EPAD = NS * EPT
NPAD = 10240
STRIP = NPAD // NS
BN = 1000


def _tc_in_body(dh, x_ref, w_ref, al_ref, ar_ref, f_ref, e_ref):
    f = jnp.dot(x_ref[...], w_ref[...], preferred_element_type=jnp.float32)
    f_ref[...] = jnp.stack([f[:, :dh], f[:, dh:]], axis=0)
    el = jnp.sum(f * al_ref[...], axis=1, keepdims=True)
    er = jnp.sum(f * ar_ref[...], axis=1, keepdims=True)
    e_ref[...] = jnp.concatenate([el, er], axis=1)


def _tc_feat_in(x, W, al, ar):
    n, din = x.shape
    dout = W.shape[1]
    dh = dout // 2
    return pl.pallas_call(
        functools.partial(_tc_in_body, dh),
        grid=(n // BN,),
        in_specs=[
            pl.BlockSpec((BN, din), lambda i: (i, 0)),
            pl.BlockSpec((din, dout), lambda i: (0, 0)),
            pl.BlockSpec((1, dout), lambda i: (0, 0)),
            pl.BlockSpec((1, dout), lambda i: (0, 0)),
        ],
        out_specs=[
            pl.BlockSpec((NC, BN, dh), lambda i: (0, i, 0)),
            pl.BlockSpec((BN, 2), lambda i: (i, 0)),
        ],
        out_shape=[
            jax.ShapeDtypeStruct((NC, n, dh), jnp.float32),
            jax.ShapeDtypeStruct((n, 2), jnp.float32),
        ],
    )(x, W, al, ar)


def _tc_mid_body(dh, a_ref, dn_ref, b_ref, w_ref, al_ref, ar_ref,
                 f_ref, e_ref):
    s = jnp.concatenate([a_ref[0], a_ref[1]], axis=1)
    dn = dn_ref[0]
    h = s / (dn + 1e-9) + b_ref[...]
    f = jnp.dot(h, w_ref[...], preferred_element_type=jnp.float32)
    f_ref[...] = jnp.stack([f[:, :dh], f[:, dh:]], axis=0)
    el = jnp.sum(f * al_ref[...], axis=1, keepdims=True)
    er = jnp.sum(f * ar_ref[...], axis=1, keepdims=True)
    e_ref[...] = jnp.concatenate([el, er], axis=1)


def _tc_feat_mid(acc, den, b, W, al, ar):
    d1h = acc.shape[2]
    dout = W.shape[1]
    dh = dout // 2
    return pl.pallas_call(
        functools.partial(_tc_mid_body, dh),
        grid=(N // BN,),
        in_specs=[
            pl.BlockSpec((NC, BN, d1h), lambda i: (0, i, 0)),
            pl.BlockSpec((NC, BN, 1), lambda i: (0, i, 0)),
            pl.BlockSpec((1, 2 * d1h), lambda i: (0, 0)),
            pl.BlockSpec((2 * d1h, dout), lambda i: (0, 0)),
            pl.BlockSpec((1, dout), lambda i: (0, 0)),
            pl.BlockSpec((1, dout), lambda i: (0, 0)),
        ],
        out_specs=[
            pl.BlockSpec((NC, BN, dh), lambda i: (0, i, 0)),
            pl.BlockSpec((BN, 2), lambda i: (i, 0)),
        ],
        out_shape=[
            jax.ShapeDtypeStruct((NC, N, dh), jnp.float32),
            jax.ShapeDtypeStruct((N, 2), jnp.float32),
        ],
    )(acc, den, b, W, al, ar)


def _tc_out_body(a_ref, dn_ref, b_ref, o_ref):
    s = jnp.concatenate([a_ref[0], a_ref[1]], axis=1)
    dn = dn_ref[0]
    h = s / (dn + 1e-9) + b_ref[...]
    m = jnp.max(h, axis=1, keepdims=True)
    ex = jnp.exp(h - m)
    o_ref[...] = ex / jnp.sum(ex, axis=1, keepdims=True)


def _tc_out(acc, den, b):
    dho = acc.shape[2]
    return pl.pallas_call(
        _tc_out_body,
        grid=(N // BN,),
        in_specs=[
            pl.BlockSpec((NC, BN, dho), lambda i: (0, i, 0)),
            pl.BlockSpec((NC, BN, 1), lambda i: (0, i, 0)),
            pl.BlockSpec((1, 2 * dho), lambda i: (0, 0)),
        ],
        out_specs=pl.BlockSpec((BN, 2 * dho), lambda i: (i, 0)),
        out_shape=jax.ShapeDtypeStruct((N, 2 * dho), jnp.float32),
    )(acc, den, b)


def _sc_body(dh, feata_hbm, featb_hbm, el_hbm, er_hbm, src_hbm, dst_hbm,
             acc_out, den_out, el_v, er_v, src_v, dst_v, ex_v, rows0_v,
             rows1_v, sbuf_v, acc_s, den_s, sem0, sem1):
    cid = lax.axis_index("c")
    sid = lax.axis_index("s")

    pltpu.sync_copy(el_hbm, el_v)
    pltpu.sync_copy(er_hbm, er_v)
    pltpu.sync_copy(src_hbm.at[sid], src_v)
    pltpu.sync_copy(dst_hbm.at[sid], dst_v)

    zero16 = jnp.zeros((16,), jnp.float32)

    @pl.loop(0, CHUNK)
    def _(e):
        for q in range(dh // 16):
            sbuf_v[e, pl.ds(q * 16, 16)] = zero16

    for q in range(CHUNK // 16):
        ex_v[pl.ds(q * 16, 16)] = zero16

    base = sid * STRIP
    for i in range(STRIP // CHUNK):
        pltpu.sync_copy(sbuf_v, acc_s.at[pl.ds(base + i * CHUNK, CHUNK)])
        pltpu.sync_copy(ex_v, den_s.at[pl.ds(base + i * CHUNK, CHUNK)])
    plsc.subcore_barrier()

    iota16 = lax.iota(jnp.int32, 16)
    gbase = sid * EPT

    def issue_gather(j, gbuf, sem):
        @pl.when(cid == 0)
        def _():
            pltpu.async_copy(feata_hbm.at[src_v.at[j]], gbuf, sem)

        @pl.when(cid != 0)
        def _():
            pltpu.async_copy(featb_hbm.at[src_v.at[j]], gbuf, sem)

    def compute_ex(j):
        @pl.loop(0, CHUNK, step=16)
        def _(k):
            s16 = src_v[j, pl.ds(k, 16)]
            d16 = dst_v[j, pl.ds(k, 16)]
            e16 = plsc.load_gather(el_v, [s16]) + plsc.load_gather(er_v, [d16])
            e16 = jnp.where(e16 >= 0.0, e16, e16 * 0.2)
            gid = gbase + j * CHUNK + k + iota16
            ex_v[pl.ds(k, 16)] = jnp.where(gid < E, jnp.exp(e16), 0.0)

    def wait_gather(gbuf, sem):
        pltpu.make_async_copy(feata_hbm.at[pl.ds(0, CHUNK)], gbuf, sem).wait()

    def scale(gbuf):
        @pl.loop(0, CHUNK, step=16)
        def _(k):
            w16 = ex_v[pl.ds(k, 16)]
            for i in range(16):
                w = w16[i]
                for q in range(dh // 16):
                    sl = pl.ds(q * 16, 16)
                    sbuf_v[k + i, sl] = gbuf[k + i, sl] * w

    def scatter(j):
        pltpu.sync_copy(sbuf_v, acc_s.at[dst_v.at[j]], add=True)
        pltpu.sync_copy(ex_v, den_s.at[dst_v.at[j]], add=True)

    issue_gather(0, rows0_v, sem0)
    issue_gather(1, rows1_v, sem1)

    @pl.loop(0, NCH - 1, step=2)
    def _(j):
        compute_ex(j)
        wait_gather(rows0_v, sem0)
        scale(rows0_v)
        issue_gather(j + 2, rows0_v, sem0)
        scatter(j)

        compute_ex(j + 1)
        wait_gather(rows1_v, sem1)
        scale(rows1_v)
        issue_gather(j + 3, rows1_v, sem1)
        scatter(j + 1)

    compute_ex(NCH - 1)
    wait_gather(rows0_v, sem0)
    scale(rows0_v)
    scatter(NCH - 1)
    wait_gather(rows1_v, sem1)

    plsc.subcore_barrier()
    pltpu.sync_copy(acc_s.at[pl.ds(base, STRIP)],
                    acc_out.at[cid, pl.ds(base, STRIP)])
    pltpu.sync_copy(den_s.at[pl.ds(base, STRIP)],
                    den_out.at[cid, pl.ds(base, STRIP)])


def _sc_layer(feata, featb, el, er, src_t, dst_t):
    dh = feata.shape[1]
    mesh = plsc.VectorSubcoreMesh(core_axis_name="c", subcore_axis_name="s")
    cp = pltpu.CompilerParams(use_tc_tiling_on_sc=False)
    if "needs_layout_passes" in pltpu.CompilerParams.__dataclass_fields__:
        cp = dataclasses.replace(cp, needs_layout_passes=False)
    kern = pl.kernel(
        functools.partial(_sc_body, dh),
        compiler_params=cp,
        out_type=(jax.ShapeDtypeStruct((NC, NPAD, dh), jnp.float32),
                  jax.ShapeDtypeStruct((NC, NPAD), jnp.float32)),
        mesh=mesh,
        scratch_types=[
            pltpu.VMEM((N,), jnp.float32),
            pltpu.VMEM((N,), jnp.float32),
            pltpu.VMEM((NCH + 1, CHUNK), jnp.int32),
            pltpu.VMEM((NCH, CHUNK), jnp.int32),
            pltpu.VMEM((CHUNK,), jnp.float32),
            pltpu.VMEM((CHUNK, dh), jnp.float32),
            pltpu.VMEM((CHUNK, dh), jnp.float32),
            pltpu.VMEM((CHUNK, dh), jnp.float32),
            pltpu.VMEM_SHARED((NPAD, dh), jnp.float32),
            pltpu.VMEM_SHARED((NPAD,), jnp.float32),
            pltpu.SemaphoreType.DMA,
            pltpu.SemaphoreType.DMA,
        ],
    )
    return kern(feata, featb, el, er, src_t, dst_t)


def kernel(x, edge_index, W1, attn_l1, attn_r1, b1, W2, attn_l2, attn_r2, b2):
    src_t = jnp.pad(jnp.pad(edge_index[0], (0, EPAD - E)).reshape(NS, NCH, CHUNK),
                    ((0, 0), (0, 1), (0, 0)))
    dst_t = jnp.pad(edge_index[1], (0, EPAD - E)).reshape(NS, NCH, CHUNK)
    fs1, eler1 = _tc_feat_in(x, W1, attn_l1.reshape(1, -1),
                             attn_r1.reshape(1, -1))
    acc1, den1 = _sc_layer(fs1[0], fs1[1], eler1[:, 0], eler1[:, 1],
                           src_t, dst_t)
    fs2, eler2 = _tc_feat_mid(acc1, den1.reshape(NC, NPAD, 1), b1.reshape(1, -1), W2,
                              attn_l2.reshape(1, -1), attn_r2.reshape(1, -1))
    acc2, den2 = _sc_layer(fs2[0], fs2[1], eler2[:, 0], eler2[:, 1],
                           src_t, dst_t)
    return _tc_out(acc2, den2.reshape(NC, NPAD, 1), b2.reshape(1, -1))


# speedup vs baseline: 1.6619x; 1.0298x over previous
import dataclasses
import functools

import jax
import jax.numpy as jnp
from jax import lax
from jax.experimental import pallas as pl
from jax.experimental.pallas import tpu as pltpu
from jax.experimental.pallas import tpu_sc as plsc

N = 10000
E = 320000
NC = 2
NS = 16
CHUNK = 128
NCH = 157
EPT = NCH * CHUNK
EPAD = NS * EPT
NPAD = 10240
STRIP = NPAD // NS
BN = 1000


def _tc_in_body(dh, x_ref, w_ref, al_ref, ar_ref, f_ref, el_ref, er_ref):
    f = jnp.dot(x_ref[...], w_ref[...], preferred_element_type=jnp.float32)
    f_ref[...] = jnp.stack([f[:, :dh], f[:, dh:]], axis=0)
    el_ref[...] = jnp.sum(f * al_ref[...], axis=1, keepdims=True)
    er_ref[...] = jnp.sum(f * ar_ref[...], axis=1, keepdims=True)


def _tc_feat_in(x, W, al, ar):
    n, din = x.shape
    dout = W.shape[1]
    dh = dout // 2
    return pl.pallas_call(
        functools.partial(_tc_in_body, dh),
        grid=(n // BN,),
        in_specs=[
            pl.BlockSpec((BN, din), lambda i: (i, 0)),
            pl.BlockSpec((din, dout), lambda i: (0, 0)),
            pl.BlockSpec((1, dout), lambda i: (0, 0)),
            pl.BlockSpec((1, dout), lambda i: (0, 0)),
        ],
        out_specs=[
            pl.BlockSpec((NC, BN, dh), lambda i: (0, i, 0)),
            pl.BlockSpec((BN, 1), lambda i: (i, 0)),
            pl.BlockSpec((BN, 1), lambda i: (i, 0)),
        ],
        out_shape=[
            jax.ShapeDtypeStruct((NC, n, dh), jnp.float32),
            jax.ShapeDtypeStruct((n, 1), jnp.float32),
            jax.ShapeDtypeStruct((n, 1), jnp.float32),
        ],
    )(x, W, al, ar)


def _tc_mid_body(dh, a_ref, dn_ref, b_ref, w_ref, al_ref, ar_ref,
                 f_ref, el_ref, er_ref):
    s = jnp.concatenate([a_ref[0], a_ref[1]], axis=1)
    dn = dn_ref[0]
    h = s / (dn + 1e-9) + b_ref[...]
    f = jnp.dot(h, w_ref[...], preferred_element_type=jnp.float32)
    f_ref[...] = jnp.stack([f[:, :dh], f[:, dh:]], axis=0)
    el_ref[...] = jnp.sum(f * al_ref[...], axis=1, keepdims=True)
    er_ref[...] = jnp.sum(f * ar_ref[...], axis=1, keepdims=True)


def _tc_feat_mid(acc, den, b, W, al, ar):
    d1h = acc.shape[2]
    dout = W.shape[1]
    dh = dout // 2
    return pl.pallas_call(
        functools.partial(_tc_mid_body, dh),
        grid=(N // BN,),
        in_specs=[
            pl.BlockSpec((NC, BN, d1h), lambda i: (0, i, 0)),
            pl.BlockSpec((NC, BN, 1), lambda i: (0, i, 0)),
            pl.BlockSpec((1, 2 * d1h), lambda i: (0, 0)),
            pl.BlockSpec((2 * d1h, dout), lambda i: (0, 0)),
            pl.BlockSpec((1, dout), lambda i: (0, 0)),
            pl.BlockSpec((1, dout), lambda i: (0, 0)),
        ],
        out_specs=[
            pl.BlockSpec((NC, BN, dh), lambda i: (0, i, 0)),
            pl.BlockSpec((BN, 1), lambda i: (i, 0)),
            pl.BlockSpec((BN, 1), lambda i: (i, 0)),
        ],
        out_shape=[
            jax.ShapeDtypeStruct((NC, N, dh), jnp.float32),
            jax.ShapeDtypeStruct((N, 1), jnp.float32),
            jax.ShapeDtypeStruct((N, 1), jnp.float32),
        ],
    )(acc, den, b, W, al, ar)


def _tc_out_body(a_ref, dn_ref, b_ref, o_ref):
    s = jnp.concatenate([a_ref[0], a_ref[1]], axis=1)
    dn = dn_ref[0]
    h = s / (dn + 1e-9) + b_ref[...]
    m = jnp.max(h, axis=1, keepdims=True)
    ex = jnp.exp(h - m)
    o_ref[...] = ex / jnp.sum(ex, axis=1, keepdims=True)


def _tc_out(acc, den, b):
    dho = acc.shape[2]
    return pl.pallas_call(
        _tc_out_body,
        grid=(N // BN,),
        in_specs=[
            pl.BlockSpec((NC, BN, dho), lambda i: (0, i, 0)),
            pl.BlockSpec((NC, BN, 1), lambda i: (0, i, 0)),
            pl.BlockSpec((1, 2 * dho), lambda i: (0, 0)),
        ],
        out_specs=pl.BlockSpec((BN, 2 * dho), lambda i: (i, 0)),
        out_shape=jax.ShapeDtypeStruct((N, 2 * dho), jnp.float32),
    )(acc, den, b)


def _sc_body(dh, fs_hbm, el_hbm, er_hbm, src_hbm, dst_hbm,
             acc_out, den_out, el_v, er_v, src_v, dst_v, ex_v, rows0_v,
             rows1_v, sbuf_v, acc_s, den_s, sem0, sem1):
    cid = lax.axis_index("c")
    sid = lax.axis_index("s")

    pltpu.sync_copy(el_hbm, el_v)
    pltpu.sync_copy(er_hbm, er_v)
    pltpu.sync_copy(src_hbm.at[sid], src_v)
    pltpu.sync_copy(dst_hbm.at[sid], dst_v)

    zero16 = jnp.zeros((16,), jnp.float32)

    @pl.loop(0, CHUNK)
    def _(e):
        for q in range(dh // 16):
            sbuf_v[e, pl.ds(q * 16, 16)] = zero16

    for q in range(CHUNK // 16):
        ex_v[pl.ds(q * 16, 16)] = zero16

    base = sid * STRIP
    for i in range(STRIP // CHUNK):
        pltpu.sync_copy(sbuf_v, acc_s.at[pl.ds(base + i * CHUNK, CHUNK)])
        pltpu.sync_copy(ex_v, den_s.at[pl.ds(base + i * CHUNK, CHUNK)])
    plsc.subcore_barrier()

    iota16 = lax.iota(jnp.int32, 16)
    gbase = sid * EPT

    def issue_gather(j, gbuf, sem):
        @pl.when(cid == 0)
        def _():
            pltpu.async_copy(fs_hbm.at[0].at[src_v.at[j]], gbuf, sem)

        @pl.when(cid != 0)
        def _():
            pltpu.async_copy(fs_hbm.at[1].at[src_v.at[j]], gbuf, sem)

    def compute_ex(j):
        @pl.loop(0, CHUNK, step=16)
        def _(k):
            s16 = src_v[j, pl.ds(k, 16)]
            d16 = dst_v[j, pl.ds(k, 16)]
            e16 = plsc.load_gather(el_v, [s16]) + plsc.load_gather(er_v, [d16])
            e16 = jnp.where(e16 >= 0.0, e16, e16 * 0.2)
            gid = gbase + j * CHUNK + k + iota16
            ex_v[pl.ds(k, 16)] = jnp.where(gid < E, jnp.exp(e16), 0.0)

    def wait_gather(gbuf, sem):
        pltpu.make_async_copy(fs_hbm.at[0].at[pl.ds(0, CHUNK)], gbuf, sem).wait()

    def scale(gbuf):
        @pl.loop(0, CHUNK, step=16)
        def _(k):
            w16 = ex_v[pl.ds(k, 16)]
            for i in range(16):
                w = w16[i]
                for q in range(dh // 16):
                    sl = pl.ds(q * 16, 16)
                    sbuf_v[k + i, sl] = gbuf[k + i, sl] * w

    def scatter(j):
        pltpu.sync_copy(sbuf_v, acc_s.at[dst_v.at[j]], add=True)
        pltpu.sync_copy(ex_v, den_s.at[dst_v.at[j]], add=True)

    issue_gather(0, rows0_v, sem0)
    issue_gather(1, rows1_v, sem1)

    @pl.loop(0, NCH - 1, step=2)
    def _(j):
        compute_ex(j)
        wait_gather(rows0_v, sem0)
        scale(rows0_v)
        issue_gather(j + 2, rows0_v, sem0)
        scatter(j)

        compute_ex(j + 1)
        wait_gather(rows1_v, sem1)
        scale(rows1_v)
        issue_gather(j + 3, rows1_v, sem1)
        scatter(j + 1)

    compute_ex(NCH - 1)
    wait_gather(rows0_v, sem0)
    scale(rows0_v)
    scatter(NCH - 1)
    wait_gather(rows1_v, sem1)

    plsc.subcore_barrier()
    pltpu.sync_copy(acc_s.at[pl.ds(base, STRIP)],
                    acc_out.at[cid, pl.ds(base, STRIP)])
    pltpu.sync_copy(den_s.at[pl.ds(base, STRIP)],
                    den_out.at[cid, pl.ds(base, STRIP)])


def _sc_layer(fs, el, er, src_t, dst_t):
    dh = fs.shape[2]
    mesh = plsc.VectorSubcoreMesh(core_axis_name="c", subcore_axis_name="s")
    cp = pltpu.CompilerParams(use_tc_tiling_on_sc=False)
    if "needs_layout_passes" in pltpu.CompilerParams.__dataclass_fields__:
        cp = dataclasses.replace(cp, needs_layout_passes=False)
    kern = pl.kernel(
        functools.partial(_sc_body, dh),
        compiler_params=cp,
        out_type=(jax.ShapeDtypeStruct((NC, NPAD, dh), jnp.float32),
                  jax.ShapeDtypeStruct((NC, NPAD), jnp.float32)),
        mesh=mesh,
        scratch_types=[
            pltpu.VMEM((N,), jnp.float32),
            pltpu.VMEM((N,), jnp.float32),
            pltpu.VMEM((NCH + 1, CHUNK), jnp.int32),
            pltpu.VMEM((NCH, CHUNK), jnp.int32),
            pltpu.VMEM((CHUNK,), jnp.float32),
            pltpu.VMEM((CHUNK, dh), jnp.float32),
            pltpu.VMEM((CHUNK, dh), jnp.float32),
            pltpu.VMEM((CHUNK, dh), jnp.float32),
            pltpu.VMEM_SHARED((NPAD, dh), jnp.float32),
            pltpu.VMEM_SHARED((NPAD,), jnp.float32),
            pltpu.SemaphoreType.DMA,
            pltpu.SemaphoreType.DMA,
        ],
    )
    return kern(fs, el, er, src_t, dst_t)


def kernel(x, edge_index, W1, attn_l1, attn_r1, b1, W2, attn_l2, attn_r2, b2):
    src_t = jnp.pad(jnp.pad(edge_index[0], (0, EPAD - E)).reshape(NS, NCH, CHUNK),
                    ((0, 0), (0, 1), (0, 0)))
    dst_t = jnp.pad(edge_index[1], (0, EPAD - E)).reshape(NS, NCH, CHUNK)
    fs1, el1, er1 = _tc_feat_in(x, W1, attn_l1.reshape(1, -1),
                                attn_r1.reshape(1, -1))
    acc1, den1 = _sc_layer(fs1, el1.reshape(N), er1.reshape(N), src_t, dst_t)
    fs2, el2, er2 = _tc_feat_mid(acc1, den1.reshape(NC, NPAD, 1), b1.reshape(1, -1), W2,
                                 attn_l2.reshape(1, -1), attn_r2.reshape(1, -1))
    acc2, den2 = _sc_layer(fs2, el2.reshape(N), er2.reshape(N), src_t, dst_t)
    return _tc_out(acc2, den2.reshape(NC, NPAD, 1), b2.reshape(1, -1))
